# Initial kernel scaffold; baseline (speedup 1.0000x reference)
import jax
import jax.numpy as jnp
from jax.experimental import pallas as pl


def kernel(z_e, emb_weight):
    raise NotImplementedError("write your pallas kernel here")


# speedup vs baseline: 1.0956x; 1.0956x over previous
import jax
import jax.numpy as jnp
from jax.experimental import pallas as pl
from jax.experimental.pallas import tpu as pltpu

_E = 1024
_D = 64


def _vq_block(z_ref, emb_ref, zsq_ref, esq_ref, zq_ref, idx_ref, loss_ref):
    i = pl.program_id(0)
    z = z_ref[...]
    emb = emb_ref[...]
    rb = z.shape[0]

    zsq = zsq_ref[...]
    esq = esq_ref[...]
    s = jax.lax.dot_general(z, emb, (((1,), (1,)), ((), ())),
                            preferred_element_type=jnp.float32)
    d = (zsq + esq) - 2.0 * s
    m = jnp.min(d, axis=1, keepdims=True)
    io = jax.lax.broadcasted_iota(jnp.int32, (rb, _E), 1)
    idx = jnp.min(jnp.where(d == m, io, _E), axis=1).astype(jnp.int32)

    oh = (jax.lax.broadcasted_iota(jnp.int32, (rb, _E), 1)
          == idx[:, None]).astype(jnp.float32)
    zq = jax.lax.dot_general(oh, emb, (((1,), (0,)), ((), ())),
                             preferred_element_type=jnp.float32)

    zq_ref[...] = z + (zq - z)
    idx_ref[0, 0, :] = idx
    diff = zq - z
    part = jnp.sum(diff * diff).reshape(1, 1)

    @pl.when(i == 0)
    def _():
        loss_ref[...] = part

    @pl.when(i > 0)
    def _():
        loss_ref[...] = loss_ref[...] + part


def kernel(z_e, emb_weight):
    B, D, T = z_e.shape
    N = B * T
    RB = 768
    NB = N // RB

    z_flat = jnp.transpose(z_e.astype(jnp.float32), (0, 2, 1)).reshape(N, D)
    zsq = jnp.sum(z_flat ** 2, axis=1, keepdims=True)
    esq = jnp.sum(emb_weight ** 2, axis=1)[None, :]

    zq_flat, idx3, loss = pl.pallas_call(
        _vq_block,
        grid=(NB,),
        in_specs=[
            pl.BlockSpec((RB, D), lambda i: (i, 0)),
            pl.BlockSpec((_E, D), lambda i: (0, 0)),
            pl.BlockSpec((RB, 1), lambda i: (i, 0)),
            pl.BlockSpec((1, _E), lambda i: (0, 0)),
        ],
        out_specs=[
            pl.BlockSpec((RB, D), lambda i: (i, 0)),
            pl.BlockSpec((1, 1, RB), lambda i: (i, 0, 0)),
            pl.BlockSpec((1, 1), lambda i: (0, 0)),
        ],
        out_shape=[
            jax.ShapeDtypeStruct((N, D), jnp.float32),
            jax.ShapeDtypeStruct((NB, 1, RB), jnp.int32),
            jax.ShapeDtypeStruct((1, 1), jnp.float32),
        ],
    )(z_flat, emb_weight, zsq, esq)

    z_q_out = jnp.transpose(zq_flat.reshape(B, T, D), (0, 2, 1)).astype(z_e.dtype)
    e_loss = (loss[0, 0] / (B * D * T)).astype(jnp.float32)
    encoding_indices = idx3.reshape(B, T)
    return (z_q_out, e_loss, encoding_indices)


# speedup vs baseline: 1.7759x; 1.6210x over previous
import jax
import jax.numpy as jnp
from jax.experimental import pallas as pl
from jax.experimental.pallas import tpu as pltpu

_E = 1024


def _vq_block(z_ref, emb_ref, esq_ref, zq_ref, idx_ref, loss_ref):
    i = pl.program_id(0)
    z = z_ref[0]
    emb = emb_ref[...]
    esq = esq_ref[...]
    t = z.shape[1]

    zsq = jnp.sum(z * z, axis=0, keepdims=True)
    s = jax.lax.dot_general(emb, z, (((1,), (0,)), ((), ())),
                            preferred_element_type=jnp.float32)
    d = (zsq + esq) - 2.0 * s
    m = jnp.min(d, axis=0, keepdims=True)
    io = jax.lax.broadcasted_iota(jnp.int32, (_E, t), 0)
    idx = jnp.min(jnp.where(d == m, io, _E), axis=0).astype(jnp.int32)

    oh = (io == idx[None, :]).astype(jnp.float32)
    zq = jax.lax.dot_general(emb, oh, (((0,), (0,)), ((), ())),
                             preferred_element_type=jnp.float32)

    zq_ref[0] = z + (zq - z)
    idx_ref[0, 0] = idx
    diff = zq - z
    part = jnp.sum(diff * diff).reshape(1, 1)

    @pl.when(i == 0)
    def _():
        loss_ref[...] = part

    @pl.when(i > 0)
    def _():
        loss_ref[...] = loss_ref[...] + part


def kernel(z_e, emb_weight):
    B, D, T = z_e.shape
    z32 = z_e.astype(jnp.float32)
    esq = jnp.sum(emb_weight ** 2, axis=1)[:, None]

    zq, idx3, loss = pl.pallas_call(
        _vq_block,
        grid=(B,),
        in_specs=[
            pl.BlockSpec((1, D, T), lambda i: (i, 0, 0)),
            pl.BlockSpec((_E, D), lambda i: (0, 0)),
            pl.BlockSpec((_E, 1), lambda i: (0, 0)),
        ],
        out_specs=[
            pl.BlockSpec((1, D, T), lambda i: (i, 0, 0)),
            pl.BlockSpec((1, 1, T), lambda i: (i, 0, 0)),
            pl.BlockSpec((1, 1), lambda i: (0, 0)),
        ],
        out_shape=[
            jax.ShapeDtypeStruct((B, D, T), jnp.float32),
            jax.ShapeDtypeStruct((B, 1, T), jnp.int32),
            jax.ShapeDtypeStruct((1, 1), jnp.float32),
        ],
    )(z32, emb_weight, esq)

    z_q_out = zq.astype(z_e.dtype)
    e_loss = (loss[0, 0] / (B * D * T)).astype(jnp.float32)
    encoding_indices = idx3.reshape(B, T)
    return (z_q_out, e_loss, encoding_indices)


# speedup vs baseline: 2.1077x; 1.1868x over previous
import jax
import jax.numpy as jnp
from jax.experimental import pallas as pl
from jax.experimental.pallas import tpu as pltpu

_E = 1024


def _vq_block(z_ref, emb_ref, emb2_ref, esq_ref, zq_ref, idx_ref, loss_ref):
    i = pl.program_id(0)
    nb = pl.num_programs(0)
    z = z_ref[0]
    emb = emb_ref[...]
    emb2 = emb2_ref[...]
    esq = esq_ref[...]
    t = z.shape[1]

    zsq = jnp.sum(z * z, axis=0, keepdims=True)
    s2 = jax.lax.dot_general(emb2, z, (((1,), (0,)), ((), ())),
                             preferred_element_type=jnp.float32)
    d = (zsq + esq) - s2
    io = jax.lax.broadcasted_iota(jnp.int32, (_E, t), 0)
    v, ix = d, io
    half = _E // 2
    while half >= 8:
        vlo, vhi = v[:half], v[half:]
        ilo, ihi = ix[:half], ix[half:]
        take_hi = vhi < vlo
        v = jnp.where(take_hi, vhi, vlo)
        ix = jnp.where(take_hi, ihi, ilo)
        half //= 2
    m = jnp.min(v, axis=0, keepdims=True)
    idx = jnp.min(jnp.where(v == m, ix, _E), axis=0).astype(jnp.int32)

    oh = (io == idx[None, :]).astype(jnp.float32)
    zq = jax.lax.dot_general(emb, oh, (((0,), (0,)), ((), ())),
                             preferred_element_type=jnp.float32)

    zq_ref[0] = z + (zq - z)
    idx_ref[0, 0] = idx
    diff = zq - z
    part = jnp.sum(diff * diff).reshape(1, 1)

    @pl.when(i == 0)
    def _():
        loss_ref[...] = part

    @pl.when(i > 0)
    def _():
        loss_ref[...] = loss_ref[...] + part

    @pl.when(i == nb - 1)
    def _():
        loss_ref[...] = loss_ref[...] / (nb * z.shape[0] * t)


def kernel(z_e, emb_weight):
    B, D, T = z_e.shape
    z32 = z_e.astype(jnp.float32)
    esq = jnp.sum(emb_weight ** 2, axis=1)[:, None]
    emb2 = emb_weight * 2.0

    zq, idx3, loss = pl.pallas_call(
        _vq_block,
        grid=(B,),
        in_specs=[
            pl.BlockSpec((1, D, T), lambda i: (i, 0, 0)),
            pl.BlockSpec((_E, D), lambda i: (0, 0)),
            pl.BlockSpec((_E, D), lambda i: (0, 0)),
            pl.BlockSpec((_E, 1), lambda i: (0, 0)),
        ],
        out_specs=[
            pl.BlockSpec((1, D, T), lambda i: (i, 0, 0)),
            pl.BlockSpec((1, 1, T), lambda i: (i, 0, 0)),
            pl.BlockSpec((1, 1), lambda i: (0, 0)),
        ],
        out_shape=[
            jax.ShapeDtypeStruct((B, D, T), jnp.float32),
            jax.ShapeDtypeStruct((B, 1, T), jnp.int32),
            jax.ShapeDtypeStruct((1, 1), jnp.float32),
        ],
    )(z32, emb_weight, emb2, esq)

    z_q_out = zq.astype(z_e.dtype)
    e_loss = loss[0, 0]
    encoding_indices = idx3.reshape(B, T)
    return (z_q_out, e_loss, encoding_indices)


# speedup vs baseline: 2.1513x; 1.0207x over previous
import jax
import jax.numpy as jnp
from jax.experimental import pallas as pl
from jax.experimental.pallas import tpu as pltpu

_E = 1024
_BB = 4


def _vq_block(z_ref, emb_ref, emb2_ref, esq_ref, zq_ref, idx_ref, loss_ref):
    i = pl.program_id(0)
    nb = pl.num_programs(0)
    emb = emb_ref[...]
    emb2 = emb2_ref[...]
    esq = esq_ref[...]

    part = jnp.zeros((1, 1), jnp.float32)
    for b in range(_BB):
        z = z_ref[b]
        t = z.shape[1]
        zsq = jnp.sum(z * z, axis=0, keepdims=True)
        s2 = jax.lax.dot_general(emb2, z, (((1,), (0,)), ((), ())),
                                 preferred_element_type=jnp.float32)
        d = (zsq + esq) - s2
        m = jnp.min(d, axis=0, keepdims=True)
        io = jax.lax.broadcasted_iota(jnp.int32, (_E, t), 0)
        idx = jnp.min(jnp.where(d == m, io, _E), axis=0).astype(jnp.int32)

        oh = (io == idx[None, :]).astype(jnp.float32)
        zq = jax.lax.dot_general(emb, oh, (((0,), (0,)), ((), ())),
                                 preferred_element_type=jnp.float32)

        zq_ref[b] = z + (zq - z)
        idx_ref[0, b] = idx
        diff = zq - z
        part = part + jnp.sum(diff * diff).reshape(1, 1)

    @pl.when(i == 0)
    def _():
        loss_ref[...] = part

    @pl.when(i > 0)
    def _():
        loss_ref[...] = loss_ref[...] + part

    @pl.when(i == nb - 1)
    def _():
        loss_ref[...] = loss_ref[...] / (nb * _BB * 64 * 576)


def kernel(z_e, emb_weight):
    B, D, T = z_e.shape
    z32 = z_e.astype(jnp.float32)
    esq = jnp.sum(emb_weight ** 2, axis=1)[:, None]
    emb2 = emb_weight * 2.0

    zq, idx3, loss = pl.pallas_call(
        _vq_block,
        grid=(B // _BB,),
        in_specs=[
            pl.BlockSpec((_BB, D, T), lambda i: (i, 0, 0)),
            pl.BlockSpec((_E, D), lambda i: (0, 0)),
            pl.BlockSpec((_E, D), lambda i: (0, 0)),
            pl.BlockSpec((_E, 1), lambda i: (0, 0)),
        ],
        out_specs=[
            pl.BlockSpec((_BB, D, T), lambda i: (i, 0, 0)),
            pl.BlockSpec((1, _BB, T), lambda i: (i, 0, 0)),
            pl.BlockSpec((1, 1), lambda i: (0, 0)),
        ],
        out_shape=[
            jax.ShapeDtypeStruct((B, D, T), jnp.float32),
            jax.ShapeDtypeStruct((B // _BB, _BB, T), jnp.int32),
            jax.ShapeDtypeStruct((1, 1), jnp.float32),
        ],
    )(z32, emb_weight, emb2, esq)

    z_q_out = zq.astype(z_e.dtype)
    e_loss = loss[0, 0]
    encoding_indices = idx3.reshape(B, T)
    return (z_q_out, e_loss, encoding_indices)


# speedup vs baseline: 2.3436x; 1.0893x over previous
import jax
import jax.numpy as jnp
from jax.experimental import pallas as pl
from jax.experimental.pallas import tpu as pltpu

_E = 1024
_BB = 8


def _vq_block(z_ref, emb_ref, emb2_ref, esq_ref, zq_ref, idx_ref, loss_ref):
    i = pl.program_id(0)
    nb = pl.num_programs(0)
    emb = emb_ref[...]
    emb2 = emb2_ref[...]
    esq = esq_ref[...]

    part = jnp.zeros((1, 1), jnp.float32)
    for b in range(_BB):
        z = z_ref[b]
        t = z.shape[1]
        zsq = jnp.sum(z * z, axis=0, keepdims=True)
        s2 = jax.lax.dot_general(emb2, z, (((1,), (0,)), ((), ())),
                                 preferred_element_type=jnp.float32)
        d = (zsq + esq) - s2
        m = jnp.min(d, axis=0, keepdims=True)
        io = jax.lax.broadcasted_iota(jnp.int32, (_E, t), 0)
        idx = jnp.min(jnp.where(d == m, io, _E), axis=0).astype(jnp.int32)

        oh = (io == idx[None, :]).astype(jnp.float32)
        zq = jax.lax.dot_general(emb, oh, (((0,), (0,)), ((), ())),
                                 preferred_element_type=jnp.float32)

        zq_ref[b] = z + (zq - z)
        idx_ref[0, b] = idx
        diff = zq - z
        part = part + jnp.sum(diff * diff).reshape(1, 1)

    @pl.when(i == 0)
    def _():
        loss_ref[...] = part

    @pl.when(i > 0)
    def _():
        loss_ref[...] = loss_ref[...] + part

    @pl.when(i == nb - 1)
    def _():
        loss_ref[...] = loss_ref[...] / (nb * _BB * 64 * 576)


def kernel(z_e, emb_weight):
    B, D, T = z_e.shape
    z32 = z_e.astype(jnp.float32)
    esq = jnp.sum(emb_weight ** 2, axis=1)[:, None]
    emb2 = emb_weight * 2.0

    zq, idx3, loss = pl.pallas_call(
        _vq_block,
        grid=(B // _BB,),
        in_specs=[
            pl.BlockSpec((_BB, D, T), lambda i: (i, 0, 0)),
            pl.BlockSpec((_E, D), lambda i: (0, 0)),
            pl.BlockSpec((_E, D), lambda i: (0, 0)),
            pl.BlockSpec((_E, 1), lambda i: (0, 0)),
        ],
        out_specs=[
            pl.BlockSpec((_BB, D, T), lambda i: (i, 0, 0)),
            pl.BlockSpec((1, _BB, T), lambda i: (i, 0, 0)),
            pl.BlockSpec((1, 1), lambda i: (0, 0)),
        ],
        out_shape=[
            jax.ShapeDtypeStruct((B, D, T), jnp.float32),
            jax.ShapeDtypeStruct((B // _BB, _BB, T), jnp.int32),
            jax.ShapeDtypeStruct((1, 1), jnp.float32),
        ],
    )(z32, emb_weight, emb2, esq)

    z_q_out = zq.astype(z_e.dtype)
    e_loss = loss[0, 0]
    encoding_indices = idx3.reshape(B, T)
    return (z_q_out, e_loss, encoding_indices)
